# baseline (speedup 1.0000x reference)
import jax
import jax.numpy as jnp
from jax.experimental import pallas as pl

_B, _FIN, _N, _T = 8, 32, 64, 128
_HID, _OUT = 64, 128
_E = 512
_EPS = 1e-5
_G = _B * _T
_GC = 256
_L = 2 * _N


def _gelu(v):
    return 0.5 * v * (1.0 + jax.lax.erf(v * (2.0 ** -0.5)))


def _leaky(v):
    return jnp.maximum(v, 0.2 * v)


def _adj_kernel(ei_ref, adj_ref):
    src = ei_ref[0:1, :]
    dst = ei_ref[1:2, :]
    rows = jax.lax.broadcasted_iota(jnp.int32, (_N, _E), 0)
    dst_oh = (rows == dst).astype(jnp.float32)
    src_oh = (rows == src).astype(jnp.float32)
    counts = jax.lax.dot_general(
        dst_oh, src_oh, (((1,), (1,)), ((), ())),
        preferred_element_type=jnp.float32)
    eye = (jax.lax.broadcasted_iota(jnp.int32, (_N, _N), 0)
           == jax.lax.broadcasted_iota(jnp.int32, (_N, _N), 1)
           ).astype(jnp.float32)
    a = counts + eye
    adj_ref[...] = jnp.concatenate([a, a], axis=1)


def _bdot(a, b):
    return jax.lax.dot_general(
        a, b, (((2,), (1,)), ((0,), (0,))),
        preferred_element_type=jnp.float32)


def _packed_softmax(adj2, ind, ajA, aiA, ajB, aiB):
    gg = ajA.shape[0]
    ones = jnp.ones_like(ajA)
    amat = jnp.concatenate([ones, ajA, ajB], axis=2)
    aicat = jnp.concatenate([aiA, aiB], axis=1)
    bmat = jnp.concatenate([aicat, ind], axis=2)
    logit = jax.lax.dot_general(
        amat, bmat, (((2,), (2,)), ((0,), (0,))),
        preferred_element_type=jnp.float32)
    bound = _leaky(jnp.maximum(
        ajA + jnp.max(aiA, axis=1, keepdims=True),
        ajB + jnp.max(aiB, axis=1, keepdims=True)))
    ex = adj2 * jnp.exp(_leaky(logit) - bound)
    den = _bdot(ex, ind)
    return ex, 1.0 / den


def _main_kernel(x_ref, adj_ref, ew_ref, eb_ref,
                 w1_ref, s1_ref, b1_ref,
                 w2_ref, s2_ref, b2_ref,
                 w3_ref, s3_ref, b3_ref, out_ref):
    adj2 = adj_ref[...]
    lane = jax.lax.broadcasted_iota(jnp.int32, (_L, 2), 0)
    col = jax.lax.broadcasted_iota(jnp.int32, (_L, 2), 1)
    ind = ((lane < _N) == (col == 0)).astype(jnp.float32)
    ind16 = jnp.broadcast_to(ind[None], (_GC, _L, 2))
    ind8 = jnp.broadcast_to(ind[None], (_GC // 2, _L, 2))
    li = jax.lax.broadcasted_iota(jnp.int32, (1, 1, _HID), 2)

    h = x_ref[...].reshape(_GC * _N, _FIN)
    h = jnp.dot(h, ew_ref[...], preferred_element_type=jnp.float32)
    h = _gelu(h + eb_ref[...])

    def layer2h(h, w_ref, s_ref, b_ref):
        z = jnp.dot(h, w_ref[...], preferred_element_type=jnp.float32)
        zz = z.reshape(_GC, _N, _HID + 4)
        xw = zz[:, :, :_HID]
        ex, r = _packed_softmax(
            adj2, ind16,
            zz[:, :, _HID + 2:_HID + 3], zz[:, :, _HID + 0:_HID + 1],
            zz[:, :, _HID + 3:_HID + 4], zz[:, :, _HID + 1:_HID + 2])
        half = _HID // 2
        stack = jnp.concatenate(
            [jnp.where(li < half, xw, 0.0),
             jnp.where(li >= half, xw, 0.0)], axis=1)
        out = _bdot(ex, stack)
        rcat = jnp.concatenate(
            [jnp.broadcast_to(r[:, :, 0:1], (_GC, _N, half)),
             jnp.broadcast_to(r[:, :, 1:2], (_GC, _N, half))], axis=2)
        out = (out * rcat).reshape(_GC * _N, _HID)
        return _gelu(out * s_ref[...] + b_ref[...])

    h = layer2h(h, w1_ref, s1_ref, b1_ref)
    h = layer2h(h, w2_ref, s2_ref, b2_ref)

    z = jnp.dot(h, w3_ref[...], preferred_element_type=jnp.float32)
    zz = z.reshape(_GC, _N, _OUT + 2)
    xw = zz[:, :, :_OUT]
    hgc = _GC // 2
    ai = zz[:, :, _OUT:_OUT + 1]
    aj = zz[:, :, _OUT + 1:_OUT + 2]
    ex, r = _packed_softmax(adj2, ind8,
                            aj[:hgc], ai[:hgc], aj[hgc:], ai[hgc:])
    rcat = jnp.concatenate(
        [jnp.broadcast_to(r[:, :, 0:1], (hgc, _N, _N)),
         jnp.broadcast_to(r[:, :, 1:2], (hgc, _N, _N))], axis=2)
    attn = ex * rcat
    out = jnp.concatenate(
        [_bdot(attn[:, :, :_N], xw[:hgc]),
         _bdot(attn[:, :, _N:], xw[hgc:])], axis=0)
    out = _gelu(out * s3_ref[...][:, None, :] + b3_ref[...][:, None, :])
    out_ref[...] = jnp.mean(out, axis=1)


def _augment(W, a_s, a_d):
    H, C = a_s.shape
    ps = jnp.stack([W[:, h * C:(h + 1) * C] @ a_s[h] for h in range(H)], axis=1)
    pd = jnp.stack([W[:, h * C:(h + 1) * C] @ a_d[h] for h in range(H)], axis=1)
    return jnp.concatenate([W, ps, pd], axis=1)


def kernel(x, edge_index, enc_W, enc_b, bn0_g, bn0_b,
           W1, as1, ad1, b1, g1, be1,
           W2, as2, ad2, b2, g2, be2,
           W3, as3, ad3, b3, g3, be3):
    inv = 1.0 / jnp.sqrt(1.0 + _EPS)
    s0 = bn0_g * inv
    ew = enc_W.T * s0[None, :]
    eb = (enc_b * s0 + bn0_b)[None, :]
    s1 = (g1 * inv)[None, :]
    bf1 = (b1 * s1[0] + be1)[None, :]
    s2 = (g2 * inv)[None, :]
    bf2 = (b2 * s2[0] + be2)[None, :]
    s3 = (g3 * inv)[None, :]
    bf3 = (b3 * s3[0] + be3)[None, :]
    w1 = _augment(W1, as1, ad1)
    w2 = _augment(W2, as2, ad2)
    w3 = _augment(W3, as3, ad3)

    adj2 = pl.pallas_call(
        _adj_kernel,
        out_shape=jax.ShapeDtypeStruct((_N, _L), jnp.float32),
    )(edge_index)

    xt = jnp.transpose(x, (0, 3, 2, 1)).reshape(_G, _N, _FIN)

    full = lambda shape: pl.BlockSpec(shape, lambda i: (0,) * len(shape))
    out = pl.pallas_call(
        _main_kernel,
        grid=(_G // _GC,),
        in_specs=[
            pl.BlockSpec((_GC, _N, _FIN), lambda i: (i, 0, 0)),
            full((_N, _L)),
            full((_FIN, _HID)), full((1, _HID)),
            full((_HID, _HID + 4)), full((1, _HID)), full((1, _HID)),
            full((_HID, _HID + 4)), full((1, _HID)), full((1, _HID)),
            full((_HID, _OUT + 2)), full((1, _OUT)), full((1, _OUT)),
        ],
        out_specs=pl.BlockSpec((_GC, _OUT), lambda i: (i, 0)),
        out_shape=jax.ShapeDtypeStruct((_G, _OUT), jnp.float32),
    )(xt, adj2, ew, eb, w1, s1, bf1, w2, s2, bf2, w3, s3, bf3)

    return out.reshape(_B, _T, _OUT)


# speedup vs baseline: 1.3857x; 1.3857x over previous
import jax
import jax.numpy as jnp
from jax.experimental import pallas as pl

_B, _FIN, _N, _T = 8, 32, 64, 128
_HID, _OUT = 64, 128
_E = 512
_EPS = 1e-5
_G = _B * _T
_GC = 128
_L = 2 * _N


def _gelu(v):
    return 0.5 * v * (1.0 + jax.lax.erf(v * (2.0 ** -0.5)))


def _leaky(v):
    return jnp.maximum(v, 0.2 * v)


def _adj_kernel(ei_ref, adj_ref):
    src = ei_ref[0:1, :]
    dst = ei_ref[1:2, :]
    rows = jax.lax.broadcasted_iota(jnp.int32, (_N, _E), 0)
    dst_oh = (rows == dst).astype(jnp.float32)
    src_oh = (rows == src).astype(jnp.float32)
    counts = jax.lax.dot_general(
        dst_oh, src_oh, (((1,), (1,)), ((), ())),
        preferred_element_type=jnp.float32)
    eye = (jax.lax.broadcasted_iota(jnp.int32, (_N, _N), 0)
           == jax.lax.broadcasted_iota(jnp.int32, (_N, _N), 1)
           ).astype(jnp.float32)
    a = counts + eye
    adj_ref[...] = jnp.concatenate([a, a], axis=1)


def _bdot(a, b):
    return jax.lax.dot_general(
        a, b, (((2,), (1,)), ((0,), (0,))),
        preferred_element_type=jnp.float32)


def _graph_bound(aux):
    h2 = aux.shape[2]
    auxt = jnp.swapaxes(aux, 1, 2)
    m = jnp.max(auxt, axis=2, keepdims=True)
    mi = jnp.max(m[:, :h2 // 2], axis=1, keepdims=True)
    mj = jnp.max(m[:, h2 // 2:], axis=1, keepdims=True)
    return _leaky(mi + mj)


def _softmax_num(adj2, ind, amat, aicat, bound):
    bmat = jnp.concatenate([aicat, ind], axis=2)
    logit = jax.lax.dot_general(
        amat, bmat, (((2,), (2,)), ((0,), (0,))),
        preferred_element_type=jnp.float32)
    ex = adj2 * jnp.exp(_leaky(logit) - bound)
    den = _bdot(ex, ind)
    return ex, den


def _main_kernel(x_ref, adj_ref, ew_ref, eb_ref,
                 w1_ref, s1_ref, b1_ref,
                 w2_ref, s2_ref, b2_ref,
                 w3_ref, s3_ref, b3_ref, out_ref):
    adj2 = adj_ref[...]
    lane = jax.lax.broadcasted_iota(jnp.int32, (_L, 2), 0)
    col = jax.lax.broadcasted_iota(jnp.int32, (_L, 2), 1)
    ind = ((lane < _N) == (col == 0)).astype(jnp.float32)
    indc = jax.lax.broadcasted_iota(jnp.int32, (2, _HID), 1)
    indh = jnp.where((indc < _HID // 2)
                     == (jax.lax.broadcasted_iota(jnp.int32, (2, _HID), 0) == 0),
                     1.0, 0.0)
    indw = jnp.where((jax.lax.broadcasted_iota(jnp.int32, (2, _L), 1) < _N)
                     == (jax.lax.broadcasted_iota(jnp.int32, (2, _L), 0) == 0),
                     1.0, 0.0)
    indg = jnp.broadcast_to(ind[None], (_GC, _L, 2))
    indg2 = jnp.broadcast_to(ind[None], (_GC // 2, _L, 2))
    indhg = jnp.broadcast_to(indh[None], (_GC, 2, _HID))
    indwg = jnp.broadcast_to(indw[None], (_GC // 2, 2, _L))

    h = x_ref[...].reshape(_GC * _N, _FIN)
    h = jnp.dot(h, ew_ref[...], preferred_element_type=jnp.float32)
    h = _gelu(h + eb_ref[...])

    def layer2h(h, w_ref, s_ref, b_ref):
        z = jnp.dot(h, w_ref[...], preferred_element_type=jnp.float32)
        zz = z.reshape(_GC, _N, 2 * _HID + 4)
        stack = jnp.concatenate(
            [zz[:, :, :_HID], zz[:, :, _HID:2 * _HID]], axis=1)
        aux = zz[:, :, 2 * _HID:]
        bound = _graph_bound(aux)
        amat = jnp.concatenate(
            [jnp.ones((_GC, _N, 1), jnp.float32),
             aux[:, :, 2:4]], axis=2)
        aicat = jnp.concatenate(
            [aux[:, :, 0:1], aux[:, :, 1:2]], axis=1)
        ex, den = _softmax_num(adj2, indg, amat, aicat, bound)
        out = _bdot(ex, stack)
        dwide = _bdot(den, indhg)
        out = (out / dwide).reshape(_GC * _N, _HID)
        return _gelu(out * s_ref[...] + b_ref[...])

    h = layer2h(h, w1_ref, s1_ref, b1_ref)
    h = layer2h(h, w2_ref, s2_ref, b2_ref)

    z = jnp.dot(h, w3_ref[...], preferred_element_type=jnp.float32)
    zz = z.reshape(_GC, _N, _OUT + 2)
    xw = zz[:, :, :_OUT]
    hgc = _GC // 2
    ai = zz[:, :, _OUT:_OUT + 1]
    aj = zz[:, :, _OUT + 1:_OUT + 2]
    aux3 = jnp.concatenate(
        [ai[:hgc], ai[hgc:], aj[:hgc], aj[hgc:]], axis=2)
    bound = _graph_bound(aux3)
    amat = jnp.concatenate(
        [jnp.ones((hgc, _N, 1), jnp.float32), aux3[:, :, 2:4]], axis=2)
    aicat = jnp.concatenate([ai[:hgc], ai[hgc:]], axis=1)
    ex, den = _softmax_num(adj2, indg2, amat, aicat, bound)
    attn = ex / _bdot(den, indwg)
    out = jnp.concatenate(
        [_bdot(attn[:, :, :_N], xw[:hgc]),
         _bdot(attn[:, :, _N:], xw[hgc:])], axis=0)
    out = _gelu(out * s3_ref[...][:, None, :] + b3_ref[...][:, None, :])
    out_ref[...] = jnp.mean(out, axis=1)


def _augment2(W, a_s, a_d):
    H, C = a_s.shape
    zero = jnp.zeros_like(W[:, :C])
    wlo = jnp.concatenate([W[:, :C], zero], axis=1)
    whi = jnp.concatenate([zero, W[:, C:]], axis=1)
    ps = jnp.stack([W[:, h * C:(h + 1) * C] @ a_s[h] for h in range(H)], axis=1)
    pd = jnp.stack([W[:, h * C:(h + 1) * C] @ a_d[h] for h in range(H)], axis=1)
    return jnp.concatenate([wlo, whi, ps, pd], axis=1)


def _augment1(W, a_s, a_d):
    ps = (W @ a_s[0])[:, None]
    pd = (W @ a_d[0])[:, None]
    return jnp.concatenate([W, ps, pd], axis=1)


def kernel(x, edge_index, enc_W, enc_b, bn0_g, bn0_b,
           W1, as1, ad1, b1, g1, be1,
           W2, as2, ad2, b2, g2, be2,
           W3, as3, ad3, b3, g3, be3):
    inv = 1.0 / jnp.sqrt(1.0 + _EPS)
    s0 = bn0_g * inv
    ew = enc_W.T * s0[None, :]
    eb = (enc_b * s0 + bn0_b)[None, :]
    s1 = (g1 * inv)[None, :]
    bf1 = (b1 * s1[0] + be1)[None, :]
    s2 = (g2 * inv)[None, :]
    bf2 = (b2 * s2[0] + be2)[None, :]
    s3 = (g3 * inv)[None, :]
    bf3 = (b3 * s3[0] + be3)[None, :]
    w1 = _augment2(W1, as1, ad1)
    w2 = _augment2(W2, as2, ad2)
    w3 = _augment1(W3, as3, ad3)

    adj2 = pl.pallas_call(
        _adj_kernel,
        out_shape=jax.ShapeDtypeStruct((_N, _L), jnp.float32),
    )(edge_index)

    xt = jnp.transpose(x, (0, 3, 2, 1)).reshape(_G, _N, _FIN)

    full = lambda shape: pl.BlockSpec(shape, lambda i: (0,) * len(shape))
    out = pl.pallas_call(
        _main_kernel,
        grid=(_G // _GC,),
        in_specs=[
            pl.BlockSpec((_GC, _N, _FIN), lambda i: (i, 0, 0)),
            full((_N, _L)),
            full((_FIN, _HID)), full((1, _HID)),
            full((_HID, 2 * _HID + 4)), full((1, _HID)), full((1, _HID)),
            full((_HID, 2 * _HID + 4)), full((1, _HID)), full((1, _HID)),
            full((_HID, _OUT + 2)), full((1, _OUT)), full((1, _OUT)),
        ],
        out_specs=pl.BlockSpec((_GC, _OUT), lambda i: (i, 0)),
        out_shape=jax.ShapeDtypeStruct((_G, _OUT), jnp.float32),
    )(xt, adj2, ew, eb, w1, s1, bf1, w2, s2, bf2, w3, s3, bf3)

    return out.reshape(_B, _T, _OUT)
